# baseline (speedup 1.0000x reference)
import dataclasses
import functools

import jax
import jax.numpy as jnp
from jax import lax
from jax.experimental import pallas as pl
from jax.experimental.pallas import tpu as pltpu
from jax.experimental.pallas import tpu_sc as plsc

N = 10000
D = 128
E = 320000
NC = 2
NS = 16
NW = NC * NS
EPT = E // NW
WIN = 100
NWIN = EPT // WIN
NPAD = 10240
ROWS = NPAD // 16
RPT = ROWS // NS
NPT = N // NS
RBLK = 1000

_mesh = plsc.VectorSubcoreMesh(core_axis_name="c", subcore_axis_name="s")

_sc_params = pltpu.CompilerParams(needs_layout_passes=False,
                                  use_tc_tiling_on_sc=False)


NR = N // 16


@functools.partial(
    pl.kernel,
    out_type=jax.ShapeDtypeStruct((NC, NR, 16), jnp.float32),
    mesh=_mesh,
    scratch_types=[
        pltpu.VMEM((EPT,), jnp.int32),
        pltpu.VMEM((ROWS, 16), jnp.float32),
        pltpu.VMEM((ROWS // 128, 128), jnp.int32),
        pltpu.VMEM_SHARED((ROWS, 16), jnp.float32),
    ],
    compiler_params=_sc_params,
)
def _sc_deg(e_hbm, deg_out, didx, hist, iot, degsh):
    cid = lax.axis_index("c")
    sid = lax.axis_index("s")
    wid = cid * NS + sid

    z16 = jnp.zeros((16,), jnp.float32)

    @pl.loop(0, ROWS)
    def _(i):
        hist[i, :] = z16

    pltpu.sync_copy(hist.at[pl.ds(0, RPT)], degsh.at[pl.ds(sid * RPT, RPT)])

    for w in range(ROWS // 128):
        for j in range(8):
            iot[w, pl.ds(j * 16, 16)] = (
                jnp.arange(16, dtype=jnp.int32) + (w * 128 + j * 16))

    pltpu.sync_copy(e_hbm.at[1, wid], didx)

    @pl.loop(0, EPT // 16)
    def _(i):
        v = didx[pl.ds(i * 16, 16)]
        cnt, last = plsc.scan_count(v)
        row = lax.div(v, 16)
        col = lax.rem(v, 16)
        plsc.addupdate_scatter(hist, [row, col], cnt.astype(jnp.float32),
                               mask=last)

    plsc.subcore_barrier()
    for w in range(ROWS // 128):
        pltpu.sync_copy(hist.at[pl.ds(w * 128, 128)],
                        degsh.at[iot.at[w]], add=True)
    plsc.subcore_barrier()
    st = jnp.minimum(sid * RPT, NR - RPT)
    pltpu.sync_copy(degsh.at[pl.ds(st, RPT)],
                    deg_out.at[cid, pl.ds(st, RPT)])


@functools.partial(
    pl.kernel,
    out_type=jax.ShapeDtypeStruct((NC, N, D), jnp.float32),
    mesh=_mesh,
    scratch_types=[
        pltpu.VMEM((NWIN // 2, WIN), jnp.int32),
        pltpu.VMEM((NWIN // 2, WIN), jnp.int32),
        pltpu.VMEM((WIN, D), jnp.float32),
        pltpu.VMEM((WIN, D), jnp.float32),
        pltpu.VMEM((WIN, D), jnp.float32),
        pltpu.VMEM_SHARED((N, D), jnp.float32),
        pltpu.SemaphoreType.DMA,
        pltpu.SemaphoreType.DMA,
        pltpu.SemaphoreType.DMA,
        pltpu.SemaphoreType.DMA,
        pltpu.SemaphoreType.DMA,
        pltpu.SemaphoreType.DMA,
    ],
    compiler_params=_sc_params,
)
def _sc_agg(h2_hbm, e_hbm, acc_out, sidx, didx, gb0, gb1, gb2, accsh,
            gsem0, gsem1, gsem2, ssem0, ssem1, ssem2):
    cid = lax.axis_index("c")
    sid = lax.axis_index("s")
    wid = cid * NS + sid

    z16 = jnp.zeros((16,), jnp.float32)

    @pl.loop(0, WIN)
    def _(i):
        for j in range(D // 16):
            gb0[i, pl.ds(j * 16, 16)] = z16

    for k in range(NPT // WIN):
        pltpu.sync_copy(gb0, accsh.at[pl.ds(sid * NPT + k * WIN, WIN)])
    _rem = NPT % WIN
    if _rem:
        pltpu.sync_copy(
            gb0.at[pl.ds(0, _rem)],
            accsh.at[pl.ds(sid * NPT + (NPT // WIN) * WIN, _rem)])

    plsc.subcore_barrier()

    def g_start(w, buf, sem):
        pltpu.async_copy(h2_hbm.at[sidx.at[w]], buf, sem)

    def g_wait(w, buf, sem):
        pltpu.make_async_copy(h2_hbm.at[sidx.at[w]], buf, sem).wait()

    def s_start(w, buf, sem):
        pltpu.async_copy(buf, accsh.at[didx.at[w]], sem, add=True)

    def s_wait(w, buf, sem):
        pltpu.make_async_copy(buf, accsh.at[didx.at[w]], sem).wait()

    bufs = ((gb0, gsem0, ssem0), (gb1, gsem1, ssem1), (gb2, gsem2, ssem2))

    WPC = NWIN // 2
    LMAIN = WPC - (WPC % 3)
    for ch in range(2):
        pltpu.sync_copy(e_hbm.at[0, wid, pl.ds(ch * WPC, WPC)], sidx)
        pltpu.sync_copy(e_hbm.at[1, wid, pl.ds(ch * WPC, WPC)], didx)

        for k in range(3):
            g_start(k, bufs[k][0], bufs[k][1])

        @pl.loop(0, LMAIN, step=3)
        def _(w):
            for k in range(3):
                buf, gsem, ssem = bufs[k]
                g_wait(w + k, buf, gsem)
                s_start(w + k, buf, ssem)
            for k in range(3):
                buf, gsem, ssem = bufs[k]
                s_wait(w + k, buf, ssem)

                @pl.when(w + k + 3 < WPC)
                def _():
                    g_start(w + k + 3, buf, gsem)

        for t in range(LMAIN, WPC):
            buf, gsem, ssem = bufs[t - LMAIN]
            tw = jnp.int32(t)
            g_wait(tw, buf, gsem)
            s_start(tw, buf, ssem)
            s_wait(tw, buf, ssem)

    plsc.subcore_barrier()
    pltpu.sync_copy(accsh.at[pl.ds(sid * NPT, NPT)],
                    acc_out.at[cid, pl.ds(sid * NPT, NPT)])


def _scale_body(x_ref, dp_ref, x2_ref, dinv_ref):
    deg = dp_ref[0] + dp_ref[1] + 1.0
    dinv = lax.rsqrt(deg)
    dinv_ref[...] = dinv
    x2_ref[...] = x_ref[...] * dinv


def _tc_scale(x, degp):
    return pl.pallas_call(
        _scale_body,
        grid=(N // RBLK,),
        in_specs=[
            pl.BlockSpec((RBLK, D), lambda i: (i, 0)),
            pl.BlockSpec((NC, RBLK, 1), lambda i: (0, i, 0)),
        ],
        out_specs=[
            pl.BlockSpec((RBLK, D), lambda i: (i, 0)),
            pl.BlockSpec((RBLK, 1), lambda i: (i, 0)),
        ],
        out_shape=[
            jax.ShapeDtypeStruct((N, D), jnp.float32),
            jax.ShapeDtypeStruct((N, 1), jnp.float32),
        ],
    )(x, degp)


def _fin_body(acc_ref, x2_ref, dinv_ref, w_ref, b_ref, o_ref):
    t = (acc_ref[0] + acc_ref[1] + x2_ref[...]) * dinv_ref[...]
    h = jnp.dot(t, w_ref[...], preferred_element_type=jnp.float32)
    o_ref[...] = jnp.maximum(h + b_ref[...], 0.0)


def _tc_finish(accp, x2, dinv, w, b2):
    return pl.pallas_call(
        _fin_body,
        grid=(N // RBLK,),
        in_specs=[
            pl.BlockSpec((NC, RBLK, D), lambda i: (0, i, 0)),
            pl.BlockSpec((RBLK, D), lambda i: (i, 0)),
            pl.BlockSpec((RBLK, 1), lambda i: (i, 0)),
            pl.BlockSpec((D, D), lambda i: (0, 0)),
            pl.BlockSpec((1, D), lambda i: (0, 0)),
        ],
        out_specs=pl.BlockSpec((RBLK, D), lambda i: (i, 0)),
        out_shape=jax.ShapeDtypeStruct((N, D), jnp.float32),
    )(accp, x2, dinv, w, b2)


@jax.jit
def kernel(x, edge_index, W, b):
    ei = edge_index.astype(jnp.int32)
    eA = ei.reshape(2, NW, EPT)
    eB = ei.reshape(2, NW, NWIN, WIN)

    degp = _sc_deg(eA)
    degp2 = degp.reshape(NC, N, 1)
    x2, dinv = _tc_scale(x, degp2)
    accp = _sc_agg(x2, eB)
    return _tc_finish(accp, x2, dinv, W, b.reshape(1, D))


# speedup vs baseline: 1.0792x; 1.0792x over previous
import dataclasses
import functools

import jax
import jax.numpy as jnp
from jax import lax
from jax.experimental import pallas as pl
from jax.experimental.pallas import tpu as pltpu
from jax.experimental.pallas import tpu_sc as plsc

N = 10000
D = 128
E = 320000
NC = 2
NS = 16
NW = NC * NS
EPT = E // NW
WIN = 125
NWIN = EPT // WIN
NPAD = 10240
ROWS = NPAD // 16
RPT = ROWS // NS
NPT = N // NS
RBLK = 1000

_mesh = plsc.VectorSubcoreMesh(core_axis_name="c", subcore_axis_name="s")

_sc_params = pltpu.CompilerParams(needs_layout_passes=False,
                                  use_tc_tiling_on_sc=False)


NR = N // 16


@functools.partial(
    pl.kernel,
    out_type=jax.ShapeDtypeStruct((NC, NR, 16), jnp.float32),
    mesh=_mesh,
    scratch_types=[
        pltpu.VMEM((EPT,), jnp.int32),
        pltpu.VMEM((ROWS, 16), jnp.float32),
        pltpu.VMEM((ROWS // 128, 128), jnp.int32),
        pltpu.VMEM_SHARED((ROWS, 16), jnp.float32),
    ],
    compiler_params=_sc_params,
)
def _sc_deg(e_hbm, deg_out, didx, hist, iot, degsh):
    cid = lax.axis_index("c")
    sid = lax.axis_index("s")
    wid = cid * NS + sid

    z16 = jnp.zeros((16,), jnp.float32)

    @pl.loop(0, ROWS)
    def _(i):
        hist[i, :] = z16

    pltpu.sync_copy(hist.at[pl.ds(0, RPT)], degsh.at[pl.ds(sid * RPT, RPT)])

    for w in range(ROWS // 128):
        for j in range(8):
            iot[w, pl.ds(j * 16, 16)] = (
                jnp.arange(16, dtype=jnp.int32) + (w * 128 + j * 16))

    pltpu.sync_copy(e_hbm.at[1, wid], didx)

    @pl.loop(0, EPT // 16, unroll=4)
    def _(i):
        v = didx[pl.ds(i * 16, 16)]
        cnt, last = plsc.scan_count(v)
        row = lax.div(v, 16)
        col = lax.rem(v, 16)
        plsc.addupdate_scatter(hist, [row, col], cnt.astype(jnp.float32),
                               mask=last)

    plsc.subcore_barrier()
    for w in range(ROWS // 128):
        pltpu.sync_copy(hist.at[pl.ds(w * 128, 128)],
                        degsh.at[iot.at[w]], add=True)
    plsc.subcore_barrier()
    st = jnp.minimum(sid * RPT, NR - RPT)
    pltpu.sync_copy(degsh.at[pl.ds(st, RPT)],
                    deg_out.at[cid, pl.ds(st, RPT)])


@functools.partial(
    pl.kernel,
    out_type=jax.ShapeDtypeStruct((NC, N, D), jnp.float32),
    mesh=_mesh,
    scratch_types=[
        pltpu.VMEM((NWIN // 2, WIN), jnp.int32),
        pltpu.VMEM((NWIN // 2, WIN), jnp.int32),
        pltpu.VMEM((WIN, D), jnp.float32),
        pltpu.VMEM((WIN, D), jnp.float32),
        pltpu.VMEM_SHARED((N, D), jnp.float32),
        pltpu.SemaphoreType.DMA,
        pltpu.SemaphoreType.DMA,
        pltpu.SemaphoreType.DMA,
        pltpu.SemaphoreType.DMA,
    ],
    compiler_params=_sc_params,
)
def _sc_agg(h2_hbm, e_hbm, acc_out, sidx, didx, gb0, gb1, accsh,
            gsem0, gsem1, ssem0, ssem1):
    cid = lax.axis_index("c")
    sid = lax.axis_index("s")
    wid = cid * NS + sid

    z16 = jnp.zeros((16,), jnp.float32)

    @pl.loop(0, WIN)
    def _(i):
        for j in range(D // 16):
            gb0[i, pl.ds(j * 16, 16)] = z16

    for k in range(NPT // WIN):
        pltpu.sync_copy(gb0, accsh.at[pl.ds(sid * NPT + k * WIN, WIN)])
    _rem = NPT % WIN
    if _rem:
        pltpu.sync_copy(
            gb0.at[pl.ds(0, _rem)],
            accsh.at[pl.ds(sid * NPT + (NPT // WIN) * WIN, _rem)])

    plsc.subcore_barrier()

    def g_start(w, buf, sem):
        pltpu.async_copy(h2_hbm.at[sidx.at[w]], buf, sem)

    def g_wait(w, buf, sem):
        pltpu.make_async_copy(h2_hbm.at[sidx.at[w]], buf, sem).wait()

    def s_start(w, buf, sem):
        pltpu.async_copy(buf, accsh.at[didx.at[w]], sem, add=True)

    def s_wait(w, buf, sem):
        pltpu.make_async_copy(buf, accsh.at[didx.at[w]], sem).wait()

    WPC = NWIN // 2
    for ch in range(2):
        pltpu.sync_copy(e_hbm.at[0, wid, pl.ds(ch * WPC, WPC)], sidx)
        pltpu.sync_copy(e_hbm.at[1, wid, pl.ds(ch * WPC, WPC)], didx)

        g_start(0, gb0, gsem0)
        g_start(1, gb1, gsem1)

        @pl.loop(0, WPC, step=2)
        def _(w):
            g_wait(w, gb0, gsem0)
            s_start(w, gb0, ssem0)
            s_wait(w, gb0, ssem0)

            @pl.when(w + 2 < WPC)
            def _():
                g_start(w + 2, gb0, gsem0)

            g_wait(w + 1, gb1, gsem1)
            s_start(w + 1, gb1, ssem1)
            s_wait(w + 1, gb1, ssem1)

            @pl.when(w + 3 < WPC)
            def _():
                g_start(w + 3, gb1, gsem1)

    plsc.subcore_barrier()
    pltpu.sync_copy(accsh.at[pl.ds(sid * NPT, NPT)],
                    acc_out.at[cid, pl.ds(sid * NPT, NPT)])


def _scale_body(x_ref, dp_ref, x2_ref, dinv_ref):
    deg = dp_ref[0] + dp_ref[1] + 1.0
    dinv = lax.rsqrt(deg)
    dinv_ref[...] = dinv
    x2_ref[...] = x_ref[...] * dinv


def _tc_scale(x, degp):
    return pl.pallas_call(
        _scale_body,
        grid=(N // RBLK,),
        in_specs=[
            pl.BlockSpec((RBLK, D), lambda i: (i, 0)),
            pl.BlockSpec((NC, RBLK, 1), lambda i: (0, i, 0)),
        ],
        out_specs=[
            pl.BlockSpec((RBLK, D), lambda i: (i, 0)),
            pl.BlockSpec((RBLK, 1), lambda i: (i, 0)),
        ],
        out_shape=[
            jax.ShapeDtypeStruct((N, D), jnp.float32),
            jax.ShapeDtypeStruct((N, 1), jnp.float32),
        ],
    )(x, degp)


def _fin_body(acc_ref, x2_ref, dinv_ref, w_ref, b_ref, o_ref):
    t = (acc_ref[0] + acc_ref[1] + x2_ref[...]) * dinv_ref[...]
    h = jnp.dot(t, w_ref[...], preferred_element_type=jnp.float32)
    o_ref[...] = jnp.maximum(h + b_ref[...], 0.0)


def _tc_finish(accp, x2, dinv, w, b2):
    return pl.pallas_call(
        _fin_body,
        grid=(N // RBLK,),
        in_specs=[
            pl.BlockSpec((NC, RBLK, D), lambda i: (0, i, 0)),
            pl.BlockSpec((RBLK, D), lambda i: (i, 0)),
            pl.BlockSpec((RBLK, 1), lambda i: (i, 0)),
            pl.BlockSpec((D, D), lambda i: (0, 0)),
            pl.BlockSpec((1, D), lambda i: (0, 0)),
        ],
        out_specs=pl.BlockSpec((RBLK, D), lambda i: (i, 0)),
        out_shape=jax.ShapeDtypeStruct((N, D), jnp.float32),
    )(accp, x2, dinv, w, b2)


@jax.jit
def kernel(x, edge_index, W, b):
    ei = edge_index.astype(jnp.int32)
    eA = ei.reshape(2, NW, EPT)
    eB = ei.reshape(2, NW, NWIN, WIN)

    degp = _sc_deg(eA)
    degp2 = degp.reshape(NC, N, 1)
    x2, dinv = _tc_scale(x, degp2)
    accp = _sc_agg(x2, eB)
    return _tc_finish(accp, x2, dinv, W, b.reshape(1, D))


# speedup vs baseline: 1.1911x; 1.1037x over previous
import dataclasses
import functools

import jax
import jax.numpy as jnp
from jax import lax
from jax.experimental import pallas as pl
from jax.experimental.pallas import tpu as pltpu
from jax.experimental.pallas import tpu_sc as plsc

N = 10000
D = 128
E = 320000
NC = 2
NS = 16
NW = NC * NS
EPT = E // NW
WIN = 125
NWIN = EPT // WIN
NPAD = 10240
ROWS = NPAD // 16
RPT = ROWS // NS
NPT = N // NS
RBLK = 1000

_mesh = plsc.VectorSubcoreMesh(core_axis_name="c", subcore_axis_name="s")

_sc_params = pltpu.CompilerParams(needs_layout_passes=False,
                                  use_tc_tiling_on_sc=False)


NR = N // 16


@functools.partial(
    pl.kernel,
    out_type=jax.ShapeDtypeStruct((NC, NR, 16), jnp.float32),
    mesh=_mesh,
    scratch_types=[
        pltpu.VMEM((EPT,), jnp.int32),
        pltpu.VMEM((ROWS, 16), jnp.float32),
        pltpu.VMEM((ROWS // 128, 128), jnp.int32),
        pltpu.VMEM_SHARED((ROWS, 16), jnp.float32),
    ],
    compiler_params=_sc_params,
)
def _sc_deg(e_hbm, deg_out, didx, hist, iot, degsh):
    cid = lax.axis_index("c")
    sid = lax.axis_index("s")
    wid = cid * NS + sid

    z16 = jnp.zeros((16,), jnp.float32)

    @pl.loop(0, ROWS)
    def _(i):
        hist[i, :] = z16

    pltpu.sync_copy(hist.at[pl.ds(0, RPT)], degsh.at[pl.ds(sid * RPT, RPT)])

    for w in range(ROWS // 128):
        for j in range(8):
            iot[w, pl.ds(j * 16, 16)] = (
                jnp.arange(16, dtype=jnp.int32) + (w * 128 + j * 16))

    pltpu.sync_copy(e_hbm.at[1, wid], didx)

    @pl.loop(0, EPT // 16, unroll=4)
    def _(i):
        v = didx[pl.ds(i * 16, 16)]
        cnt, last = plsc.scan_count(v)
        row = lax.div(v, 16)
        col = lax.rem(v, 16)
        plsc.addupdate_scatter(hist, [row, col], cnt.astype(jnp.float32),
                               mask=last)

    plsc.subcore_barrier()
    for w in range(ROWS // 128):
        pltpu.sync_copy(hist.at[pl.ds(w * 128, 128)],
                        degsh.at[iot.at[w]], add=True)
    plsc.subcore_barrier()
    st = jnp.minimum(sid * RPT, NR - RPT)
    pltpu.sync_copy(degsh.at[pl.ds(st, RPT)],
                    deg_out.at[cid, pl.ds(st, RPT)])


@functools.partial(
    pl.kernel,
    out_type=jax.ShapeDtypeStruct((NC, N, D), jnp.float32),
    mesh=_mesh,
    scratch_types=[
        pltpu.VMEM((NWIN // 2, WIN), jnp.int32),
        pltpu.VMEM((NWIN // 2, WIN), jnp.int32),
        pltpu.VMEM((WIN, D), jnp.float32),
        pltpu.VMEM((WIN, D), jnp.float32),
        pltpu.VMEM_SHARED((N, D), jnp.float32),
        pltpu.SemaphoreType.DMA,
        pltpu.SemaphoreType.DMA,
        pltpu.SemaphoreType.DMA,
        pltpu.SemaphoreType.DMA,
    ],
    compiler_params=_sc_params,
)
def _sc_agg(h2_hbm, e_hbm, acc_out, sidx, didx, gb0, gb1, accsh,
            gsem0, gsem1, ssem0, ssem1):
    cid = lax.axis_index("c")
    sid = lax.axis_index("s")
    wid = cid * NS + sid

    z16 = jnp.zeros((16,), jnp.float32)

    @pl.loop(0, WIN)
    def _(i):
        for j in range(D // 16):
            gb0[i, pl.ds(j * 16, 16)] = z16

    for k in range(NPT // WIN):
        pltpu.sync_copy(gb0, accsh.at[pl.ds(sid * NPT + k * WIN, WIN)])
    _rem = NPT % WIN
    if _rem:
        pltpu.sync_copy(
            gb0.at[pl.ds(0, _rem)],
            accsh.at[pl.ds(sid * NPT + (NPT // WIN) * WIN, _rem)])

    plsc.subcore_barrier()

    def g_start(w, buf, sem):
        pltpu.async_copy(h2_hbm.at[sidx.at[w]], buf, sem)

    def g_wait(w, buf, sem):
        pltpu.make_async_copy(h2_hbm.at[sidx.at[w]], buf, sem).wait()

    def s_start(w, buf, sem):
        pltpu.async_copy(buf, accsh.at[didx.at[w]], sem, add=True)

    def s_wait(w, buf, sem):
        pltpu.make_async_copy(buf, accsh.at[didx.at[w]], sem).wait()

    WPC = NWIN // 2
    for ch in range(2):
        pltpu.sync_copy(e_hbm.at[0, wid, pl.ds(ch * WPC, WPC)], sidx)
        pltpu.sync_copy(e_hbm.at[1, wid, pl.ds(ch * WPC, WPC)], didx)

        g_start(0, gb0, gsem0)
        g_start(1, gb1, gsem1)

        @pl.loop(0, WPC, step=2)
        def _(w):
            g_wait(w, gb0, gsem0)
            s_start(w, gb0, ssem0)
            s_wait(w, gb0, ssem0)

            @pl.when(w + 2 < WPC)
            def _():
                g_start(w + 2, gb0, gsem0)

            g_wait(w + 1, gb1, gsem1)
            s_start(w + 1, gb1, ssem1)
            s_wait(w + 1, gb1, ssem1)

            @pl.when(w + 3 < WPC)
            def _():
                g_start(w + 3, gb1, gsem1)

    plsc.subcore_barrier()
    pltpu.sync_copy(accsh.at[pl.ds(sid * NPT, NPT)],
                    acc_out.at[cid, pl.ds(sid * NPT, NPT)])


GB = 5
GR = NR // GB


def _scale_body(x_ref, dp_ref, x2_ref):
    i = pl.program_id(0)
    dinv = lax.rsqrt(dp_ref[0, i] + dp_ref[1, i] + 1.0)
    x2_ref[...] = x_ref[...] * dinv[None, :, :, None]


def _tc_scale(x4, degp4):
    return pl.pallas_call(
        _scale_body,
        grid=(GB,),
        in_specs=[
            pl.BlockSpec((1, GR, 16, D), lambda i: (i, 0, 0, 0)),
            pl.BlockSpec((NC, GB, GR, 16), lambda i: (0, 0, 0, 0)),
        ],
        out_specs=pl.BlockSpec((1, GR, 16, D), lambda i: (i, 0, 0, 0)),
        out_shape=jax.ShapeDtypeStruct((GB, GR, 16, D), jnp.float32),
    )(x4, degp4)


def _fin_body(acc_ref, x2_ref, dp_ref, w_ref, b_ref, o_ref):
    i = pl.program_id(0)
    dinv = lax.rsqrt(dp_ref[0, i] + dp_ref[1, i] + 1.0)
    s = (acc_ref[0, 0] + acc_ref[1, 0] + x2_ref[0]) * dinv[:, :, None]
    t = s.reshape(GR * 16, D)
    h = jnp.dot(t, w_ref[...], preferred_element_type=jnp.float32)
    o_ref[...] = jnp.maximum(h + b_ref[...], 0.0)


def _tc_finish(accp5, x24, degp4, w, b2):
    return pl.pallas_call(
        _fin_body,
        grid=(GB,),
        in_specs=[
            pl.BlockSpec((NC, 1, GR, 16, D), lambda i: (0, i, 0, 0, 0)),
            pl.BlockSpec((1, GR, 16, D), lambda i: (i, 0, 0, 0)),
            pl.BlockSpec((NC, GB, GR, 16), lambda i: (0, 0, 0, 0)),
            pl.BlockSpec((D, D), lambda i: (0, 0)),
            pl.BlockSpec((1, D), lambda i: (0, 0)),
        ],
        out_specs=pl.BlockSpec((GR * 16, D), lambda i: (i, 0)),
        out_shape=jax.ShapeDtypeStruct((N, D), jnp.float32),
    )(accp5, x24, degp4, w, b2)


@jax.jit
def kernel(x, edge_index, W, b):
    ei = edge_index.astype(jnp.int32)
    eA = ei.reshape(2, NW, EPT)
    eB = ei.reshape(2, NW, NWIN, WIN)

    degp = _sc_deg(eA)
    degp4 = degp.reshape(NC, GB, GR, 16)
    x2 = _tc_scale(x.reshape(GB, GR, 16, D), degp4)
    accp = _sc_agg(x2.reshape(N, D), eB)
    return _tc_finish(accp.reshape(NC, GB, GR, 16, D),
                      x2, degp4, W, b.reshape(1, D))


# speedup vs baseline: 1.1926x; 1.0012x over previous
import dataclasses
import functools

import jax
import jax.numpy as jnp
from jax import lax
from jax.experimental import pallas as pl
from jax.experimental.pallas import tpu as pltpu
from jax.experimental.pallas import tpu_sc as plsc

N = 10000
D = 128
E = 320000
NC = 2
NS = 16
NW = NC * NS
EPT = E // NW
WIN = 125
NWIN = EPT // WIN
NPAD = 10240
ROWS = NPAD // 16
RPT = ROWS // NS
NPT = N // NS
RBLK = 1000

_mesh = plsc.VectorSubcoreMesh(core_axis_name="c", subcore_axis_name="s")

_sc_params = pltpu.CompilerParams(needs_layout_passes=False,
                                  use_tc_tiling_on_sc=False)


NR = N // 16


@functools.partial(
    pl.kernel,
    out_type=jax.ShapeDtypeStruct((NC, NR, 16), jnp.float32),
    mesh=_mesh,
    scratch_types=[
        pltpu.VMEM((EPT,), jnp.int32),
        pltpu.VMEM((ROWS, 16), jnp.float32),
        pltpu.VMEM((ROWS // 128, 128), jnp.int32),
        pltpu.VMEM_SHARED((ROWS, 16), jnp.float32),
    ],
    compiler_params=_sc_params,
)
def _sc_deg(e_hbm, deg_out, didx, hist, iot, degsh):
    cid = lax.axis_index("c")
    sid = lax.axis_index("s")
    wid = cid * NS + sid

    z16 = jnp.zeros((16,), jnp.float32)

    @pl.loop(0, ROWS)
    def _(i):
        hist[i, :] = z16

    pltpu.sync_copy(hist.at[pl.ds(0, RPT)], degsh.at[pl.ds(sid * RPT, RPT)])

    for w in range(ROWS // 128):
        for j in range(8):
            iot[w, pl.ds(j * 16, 16)] = (
                jnp.arange(16, dtype=jnp.int32) + (w * 128 + j * 16))

    pltpu.sync_copy(e_hbm.at[1, wid], didx)

    @pl.loop(0, EPT // 16, unroll=4)
    def _(i):
        v = didx[pl.ds(i * 16, 16)]
        cnt, last = plsc.scan_count(v)
        row = lax.div(v, 16)
        col = lax.rem(v, 16)
        plsc.addupdate_scatter(hist, [row, col], cnt.astype(jnp.float32),
                               mask=last)

    plsc.subcore_barrier()
    for w in range(ROWS // 128):
        pltpu.sync_copy(hist.at[pl.ds(w * 128, 128)],
                        degsh.at[iot.at[w]], add=True)
    plsc.subcore_barrier()
    st = jnp.minimum(sid * RPT, NR - RPT)
    pltpu.sync_copy(degsh.at[pl.ds(st, RPT)],
                    deg_out.at[cid, pl.ds(st, RPT)])


@functools.partial(
    pl.kernel,
    out_type=jax.ShapeDtypeStruct((NC, N, D), jnp.float32),
    mesh=_mesh,
    scratch_types=[
        pltpu.VMEM((NWIN // 2, WIN), jnp.int32),
        pltpu.VMEM((NWIN // 2, WIN), jnp.int32),
        pltpu.VMEM((WIN, D), jnp.float32),
        pltpu.VMEM((WIN, D), jnp.float32),
        pltpu.VMEM_SHARED((N, D), jnp.float32),
        pltpu.SemaphoreType.DMA,
        pltpu.SemaphoreType.DMA,
        pltpu.SemaphoreType.DMA,
        pltpu.SemaphoreType.DMA,
    ],
    compiler_params=_sc_params,
)
def _sc_agg(h2_hbm, e_hbm, acc_out, sidx, didx, gb0, gb1, accsh,
            gsem0, gsem1, ssem0, ssem1):
    cid = lax.axis_index("c")
    sid = lax.axis_index("s")
    wid = cid * NS + sid

    z16 = jnp.zeros((16,), jnp.float32)

    @pl.loop(0, WIN)
    def _(i):
        for j in range(D // 16):
            gb0[i, pl.ds(j * 16, 16)] = z16

    for k in range(NPT // WIN):
        pltpu.sync_copy(gb0, accsh.at[pl.ds(sid * NPT + k * WIN, WIN)])
    _rem = NPT % WIN
    if _rem:
        pltpu.sync_copy(
            gb0.at[pl.ds(0, _rem)],
            accsh.at[pl.ds(sid * NPT + (NPT // WIN) * WIN, _rem)])

    plsc.subcore_barrier()

    def g_start(w, buf, sem):
        pltpu.async_copy(h2_hbm.at[sidx.at[w]], buf, sem)

    def g_wait(w, buf, sem):
        pltpu.make_async_copy(h2_hbm.at[sidx.at[w]], buf, sem).wait()

    def s_start(w, buf, sem):
        pltpu.async_copy(buf, accsh.at[didx.at[w]], sem, add=True)

    def s_wait(w, buf, sem):
        pltpu.make_async_copy(buf, accsh.at[didx.at[w]], sem).wait()

    WPC = NWIN // 2
    for ch in range(2):
        pltpu.sync_copy(e_hbm.at[0, wid, pl.ds(ch * WPC, WPC)], sidx)
        pltpu.sync_copy(e_hbm.at[1, wid, pl.ds(ch * WPC, WPC)], didx)

        g_start(0, gb0, gsem0)
        g_start(1, gb1, gsem1)

        @pl.loop(0, WPC, step=2)
        def _(w):
            g_wait(w, gb0, gsem0)
            s_start(w, gb0, ssem0)
            s_wait(w, gb0, ssem0)

            @pl.when(w + 2 < WPC)
            def _():
                g_start(w + 2, gb0, gsem0)

            g_wait(w + 1, gb1, gsem1)
            s_start(w + 1, gb1, ssem1)
            s_wait(w + 1, gb1, ssem1)

            @pl.when(w + 3 < WPC)
            def _():
                g_start(w + 3, gb1, gsem1)

    plsc.subcore_barrier()
    pltpu.sync_copy(accsh.at[pl.ds(sid * NPT, NPT)],
                    acc_out.at[cid, pl.ds(sid * NPT, NPT)])


GB = 5
GR = NR // GB


def _scale_body(x_ref, dp_ref, x2_ref):
    i = pl.program_id(0)
    dinv = lax.rsqrt(dp_ref[0, i] + dp_ref[1, i] + 1.0)
    x3 = x_ref[...].reshape(GR, 16, D)
    x2_ref[...] = (x3 * dinv[:, :, None]).reshape(GR * 16, D)


def _tc_scale(x, degp4):
    return pl.pallas_call(
        _scale_body,
        grid=(GB,),
        in_specs=[
            pl.BlockSpec((GR * 16, D), lambda i: (i, 0)),
            pl.BlockSpec((NC, GB, GR, 16), lambda i: (0, 0, 0, 0)),
        ],
        out_specs=pl.BlockSpec((GR * 16, D), lambda i: (i, 0)),
        out_shape=jax.ShapeDtypeStruct((N, D), jnp.float32),
    )(x, degp4)


def _fin_body(acc_ref, x2_ref, dp_ref, w_ref, b_ref, o_ref):
    i = pl.program_id(0)
    dinv = lax.rsqrt(dp_ref[0, i] + dp_ref[1, i] + 1.0)
    s3 = (acc_ref[0] + acc_ref[1] + x2_ref[...]).reshape(GR, 16, D)
    t = (s3 * dinv[:, :, None]).reshape(GR * 16, D)
    h = jnp.dot(t, w_ref[...], preferred_element_type=jnp.float32)
    o_ref[...] = jnp.maximum(h + b_ref[...], 0.0)


def _tc_finish(accp, x2, degp4, w, b2):
    return pl.pallas_call(
        _fin_body,
        grid=(GB,),
        in_specs=[
            pl.BlockSpec((NC, GR * 16, D), lambda i: (0, i, 0)),
            pl.BlockSpec((GR * 16, D), lambda i: (i, 0)),
            pl.BlockSpec((NC, GB, GR, 16), lambda i: (0, 0, 0, 0)),
            pl.BlockSpec((D, D), lambda i: (0, 0)),
            pl.BlockSpec((1, D), lambda i: (0, 0)),
        ],
        out_specs=pl.BlockSpec((GR * 16, D), lambda i: (i, 0)),
        out_shape=jax.ShapeDtypeStruct((N, D), jnp.float32),
    )(accp, x2, degp4, w, b2)


@jax.jit
def kernel(x, edge_index, W, b):
    ei = edge_index.astype(jnp.int32)
    eA = ei.reshape(2, NW, EPT)
    eB = ei.reshape(2, NW, NWIN, WIN)

    degp = _sc_deg(eA)
    degp4 = degp.reshape(NC, GB, GR, 16)
    x2 = _tc_scale(x, degp4)
    accp = _sc_agg(x2, eB)
    return _tc_finish(accp, x2, degp4, W, b.reshape(1, D))
